# Initial kernel scaffold; baseline (speedup 1.0000x reference)
import jax
import jax.numpy as jnp
from jax.experimental import pallas as pl


def kernel(x, embeddings, alpha, conv_w, conv_b, bn_gamma, bn_beta):
    raise NotImplementedError("write your pallas kernel here")


# speedup vs baseline: 3.8846x; 3.8846x over previous
import functools

import jax
import jax.numpy as jnp
from jax import lax
from jax.experimental import pallas as pl
from jax.experimental.pallas import tpu as pltpu
from jax.experimental.pallas import tpu_sc as plsc

T_STEPS = 16
EMBED_DIM = 16
NUM_EMBED = 8192
COMMIT = 0.25
TAU_LIF = 2.0
V_TH = 1.0
TAU_S = 2.0
MEM_DECAY = 0.8

B_SZ, C_SZ, H_SZ, W_SZ = 32, 16, 32, 32
HW = H_SZ * W_SZ
N_TOK = B_SZ * HW
CODE_CHUNK = 512
N_CHUNK = NUM_EMBED // CODE_CHUNK


def _vq_kernel(x_ref, cf_ref, par_ref, e_ref, idx_ref, mo_ref):
    acc1 = jnp.zeros((C_SZ, HW), jnp.float32)
    acc2 = jnp.zeros((C_SZ, HW), jnp.float32)
    for t in range(T_STEPS):
        xt = x_ref[t, 0].reshape(C_SZ, HW)
        xb = xt.astype(jnp.bfloat16).astype(jnp.float32)
        acc1 = acc1 + cf_ref[0, t] * xb
        acc2 = acc2 + xt
    one_m_alpha = par_ref[0, 0]
    alpha = par_ref[0, 1]
    mo = one_m_alpha * acc1 + (alpha * acc2) / T_STEPS

    mo_ref[...] = mo
    mob = mo.astype(jnp.bfloat16).astype(jnp.float32)

    fsq = jnp.sum(mo * mo, axis=0, keepdims=True)

    best_m = jnp.full((1, HW), jnp.inf, dtype=jnp.float32)
    best_i = jnp.zeros((1, HW), dtype=jnp.int32)
    for c in range(N_CHUNK):
        e_blk = e_ref[pl.ds(c * CODE_CHUNK, CODE_CHUNK), :]
        e_nrm = jnp.sum(e_blk * e_blk, axis=1, keepdims=True)
        scores = (fsq + e_nrm) - 2.0 * jnp.dot(
            e_blk.astype(jnp.bfloat16), mob.astype(jnp.bfloat16),
            preferred_element_type=jnp.float32)
        m = jnp.min(scores, axis=0, keepdims=True)
        row = lax.broadcasted_iota(jnp.int32, scores.shape, 0)
        a = jnp.min(jnp.where(scores == m, row, NUM_EMBED),
                    axis=0, keepdims=True) + c * CODE_CHUNK
        upd = m < best_m
        best_i = jnp.where(upd, a, best_i)
        best_m = jnp.where(upd, m, best_m)

    idx_ref[0, 0, :] = best_i[0, :]


def _run_vq(x, cf_row, par_row, embeddings):
    return pl.pallas_call(
        _vq_kernel,
        grid=(B_SZ,),
        in_specs=[
            pl.BlockSpec((T_STEPS, 1, C_SZ, H_SZ, W_SZ),
                         lambda b: (0, b, 0, 0, 0)),
            pl.BlockSpec((1, T_STEPS), lambda b: (0, 0)),
            pl.BlockSpec((1, T_STEPS), lambda b: (0, 0)),
            pl.BlockSpec((NUM_EMBED, EMBED_DIM), lambda b: (0, 0)),
        ],
        out_specs=[
            pl.BlockSpec((1, 1, HW), lambda b: (b, 0, 0)),
            pl.BlockSpec((C_SZ, HW), lambda b: (0, b)),
        ],
        out_shape=[
            jax.ShapeDtypeStruct((B_SZ, 1, HW), jnp.int32),
            jax.ShapeDtypeStruct((C_SZ, N_TOK), jnp.float32),
        ],
    )(x, cf_row, par_row, embeddings)


def _make_sc_gather():
    info = plsc.get_sparse_core_info()
    nc, ns = info.num_cores, info.num_subcores
    nw = nc * ns
    b_per_w = N_TOK // nw
    mesh = plsc.VectorSubcoreMesh(core_axis_name="c", subcore_axis_name="s")

    @functools.partial(
        pl.kernel, mesh=mesh,
        compiler_params=pltpu.CompilerParams(use_tc_tiling_on_sc=False),
        out_type=jax.ShapeDtypeStruct((N_TOK, EMBED_DIM), jnp.float32),
        scratch_types=[
            pltpu.VMEM((b_per_w,), jnp.int32),
            pltpu.VMEM((b_per_w, EMBED_DIM), jnp.float32),
            pltpu.SemaphoreType.DMA,
        ],
    )
    def gather_k(table_hbm, idx_hbm, out_hbm, idx_v, rows_v, sem):
        wid = lax.axis_index("s") * nc + lax.axis_index("c")
        base = wid * b_per_w
        pltpu.sync_copy(idx_hbm.at[pl.ds(base, b_per_w)], idx_v)
        pltpu.async_copy(table_hbm.at[idx_v], rows_v, sem).wait()
        pltpu.sync_copy(rows_v, out_hbm.at[pl.ds(base, b_per_w)])

    return gather_k


def _stats_kernel(q_ref, mo_ref, wm_ref, sum_ref, ssq_ref):
    b = pl.program_id(0)
    mo = mo_ref[...]
    qst = mo + (q_ref[...].T - mo)
    y = jnp.dot(wm_ref[...].astype(jnp.bfloat16),
                qst.astype(jnp.bfloat16),
                preferred_element_type=jnp.float32)

    @pl.when(b == 0)
    def _():
        sum_ref[...] = jnp.zeros((1, C_SZ), jnp.float32)
        ssq_ref[...] = jnp.zeros((1, C_SZ), jnp.float32)

    sum_ref[...] += jnp.sum(y, axis=1, keepdims=True).reshape(1, C_SZ)
    ssq_ref[...] += jnp.sum(y * y, axis=1, keepdims=True).reshape(1, C_SZ)


def _run_stats(q, mo, wm):
    return pl.pallas_call(
        _stats_kernel,
        grid=(B_SZ,),
        in_specs=[
            pl.BlockSpec((HW, EMBED_DIM), lambda b: (b, 0)),
            pl.BlockSpec((C_SZ, HW), lambda b: (0, b)),
            pl.BlockSpec((C_SZ, C_SZ), lambda b: (0, 0)),
        ],
        out_specs=[
            pl.BlockSpec((1, C_SZ), lambda b: (0, 0)),
            pl.BlockSpec((1, C_SZ), lambda b: (0, 0)),
        ],
        out_shape=[
            jax.ShapeDtypeStruct((1, C_SZ), jnp.float32),
            jax.ShapeDtypeStruct((1, C_SZ), jnp.float32),
        ],
    )(q, mo, wm)


def _post_kernel(q_ref, mo_ref, x_ref, wm_ref, cb_ref, gm_ref, bt_ref,
                 sum_ref, ssq_ref, spk_ref, l1_ref, l2_ref):
    b = pl.program_id(0)
    n = jnp.float32(N_TOK)

    qt = q_ref[...].T
    mo = mo_ref[...]
    l1 = jnp.sum((qt - mo) ** 2)
    qst = mo + (qt - mo)

    y = jnp.dot(wm_ref[...].astype(jnp.bfloat16), qst.astype(jnp.bfloat16),
                preferred_element_type=jnp.float32)
    y = y + cb_ref[0, :][:, None]

    mean = sum_ref[0, :] / n + cb_ref[0, :]
    var = ssq_ref[0, :] / n - (sum_ref[0, :] / n) ** 2
    yn = ((y - mean[:, None]) / jnp.sqrt(var + 1e-5)[:, None]
          * gm_ref[0, :][:, None] + bt_ref[0, :][:, None])

    v = jnp.zeros((C_SZ, HW), jnp.float32)
    syn = jnp.zeros((C_SZ, HW), jnp.float32)
    l2 = jnp.float32(0.0)
    for t in range(T_STEPS):
        v = v + (yn - v) / TAU_LIF
        s = (v >= V_TH).astype(jnp.float32)
        v = v * (1.0 - s)
        spk_ref[t, 0] = s.reshape(C_SZ, H_SZ, W_SZ)
        d = s - x_ref[t, 0].reshape(C_SZ, HW)
        syn = syn + (d - syn) / TAU_S
        l2 = l2 + jnp.sum(syn * syn)

    @pl.when(b == 0)
    def _():
        l1_ref[...] = jnp.zeros((1, 1), jnp.float32)
        l2_ref[...] = jnp.zeros((1, 1), jnp.float32)

    l1_ref[...] += l1.reshape(1, 1)
    l2_ref[...] += l2.reshape(1, 1)


def _run_post(q, mo, x, wm, cb, gm, bt, sums, ssq):
    return pl.pallas_call(
        _post_kernel,
        grid=(B_SZ,),
        in_specs=[
            pl.BlockSpec((HW, EMBED_DIM), lambda b: (b, 0)),
            pl.BlockSpec((C_SZ, HW), lambda b: (0, b)),
            pl.BlockSpec((T_STEPS, 1, C_SZ, H_SZ, W_SZ),
                         lambda b: (0, b, 0, 0, 0)),
            pl.BlockSpec((C_SZ, C_SZ), lambda b: (0, 0)),
            pl.BlockSpec((1, C_SZ), lambda b: (0, 0)),
            pl.BlockSpec((1, C_SZ), lambda b: (0, 0)),
            pl.BlockSpec((1, C_SZ), lambda b: (0, 0)),
            pl.BlockSpec((1, C_SZ), lambda b: (0, 0)),
            pl.BlockSpec((1, C_SZ), lambda b: (0, 0)),
        ],
        out_specs=[
            pl.BlockSpec((T_STEPS, 1, C_SZ, H_SZ, W_SZ),
                         lambda b: (0, b, 0, 0, 0)),
            pl.BlockSpec((1, 1), lambda b: (0, 0)),
            pl.BlockSpec((1, 1), lambda b: (0, 0)),
        ],
        out_shape=[
            jax.ShapeDtypeStruct((T_STEPS, B_SZ, C_SZ, H_SZ, W_SZ),
                                 jnp.float32),
            jax.ShapeDtypeStruct((1, 1), jnp.float32),
            jax.ShapeDtypeStruct((1, 1), jnp.float32),
        ],
    )(q, mo, x, wm, cb, gm, bt, sums, ssq)


def kernel(x, embeddings, alpha, conv_w, conv_b, bn_gamma, bn_beta):
    t = jnp.arange(T_STEPS - 1, -1, -1, dtype=jnp.float32)
    cf_row = (jnp.power(MEM_DECAY, t).astype(jnp.bfloat16)
              .astype(jnp.float32).reshape(1, T_STEPS))
    par_row = jnp.zeros((T_STEPS,), jnp.float32)
    par_row = par_row.at[0].set(1.0 - alpha).at[1].set(alpha)
    par_row = par_row.reshape(1, T_STEPS)

    idx2d, mo = _run_vq(x, cf_row, par_row, embeddings)

    q = _make_sc_gather()(embeddings, idx2d.reshape(N_TOK))

    wm = conv_w[:, :, 0, 0]
    cb = conv_b.reshape(1, C_SZ)
    gm = bn_gamma.reshape(1, C_SZ)
    bt = bn_beta.reshape(1, C_SZ)
    sums, ssq = _run_stats(q, mo, wm)
    q_spk, l1sum, l2sum = _run_post(q, mo, x, wm, cb, gm, bt, sums, ssq)

    m1 = l1sum[0, 0] / (N_TOK * EMBED_DIM)
    m2 = l2sum[0, 0] / (T_STEPS * B_SZ * C_SZ * H_SZ * W_SZ)
    loss_1 = m1 + COMMIT * m1
    loss_2 = m2 + COMMIT * m2

    return (q_spk, loss_1 + loss_2, jnp.float32(0.0))
